# baseline (speedup 1.0000x reference)
import functools

import jax
import jax.numpy as jnp
from jax import lax
from jax.experimental import pallas as pl
from jax.experimental.pallas import tpu as pltpu
from jax.experimental.pallas import tpu_sc as plsc

N = 10000
E = 320000
G = 64
D_IN = 128

NC = 2
NS = 16
NW = NC * NS
EPW = E // NW
CHUNK = 80
NCHUNK = EPW // CHUNK
ECHUNK = 40
ENCHUNK = EPW // ECHUNK
N_PAD = 10240
RPT = N_PAD // NS

def _sc_mesh():
    return plsc.VectorSubcoreMesh(
        core_axis_name="c", subcore_axis_name="s",
        num_cores=NC, num_subcores=NS)


HROW = N_PAD // 128


def _deg_body(dst_hbm, zeros_hbm, out_hbm, didx, hist, rid, acc, sem):
    c = lax.axis_index("c")
    s = lax.axis_index("s")
    wid = s * NC + c

    @pl.when(s == 0)
    def _():
        pltpu.sync_copy(zeros_hbm, acc)

    def zrow(i, carry):
        def zc(g, carry2):
            hist[i, pl.ds(g * 16, 16)] = jnp.zeros((16,), jnp.float32)
            return carry2
        return lax.fori_loop(0, 128 // 16, zc, carry)

    lax.fori_loop(0, HROW, zrow, 0)
    pltpu.sync_copy(dst_hbm.at[pl.ds(wid * EPW, EPW)], didx)
    for g in range(HROW // 16):
        rid[pl.ds(g * 16, 16)] = (
            lax.iota(jnp.int32, 16) + jnp.int32(g * 16))
    ones16 = jnp.ones((16,), jnp.float32)

    def grp(g, carry):
        iv = didx[pl.ds(g * 16, 16)]
        row = lax.shift_right_logical(iv, 7)
        col = lax.bitwise_and(iv, jnp.int32(127))
        plsc.addupdate_scatter(hist, [row, col], ones16)
        return carry

    lax.fori_loop(0, EPW // 16, grp, 0)
    plsc.subcore_barrier()
    pltpu.sync_copy(hist, acc.at[rid], add=True)
    plsc.subcore_barrier()

    @pl.when(s == 0)
    def _():
        pltpu.sync_copy(acc, out_hbm.at[c])


NBUF = 5


def _edge_body(d, table_hbm, src_hbm, dst2_hbm, zeros_hbm, out_hbm,
               sidx, di0, di1, di2, di3, di4,
               rows0, rows1, rows2, rows3, rows4,
               acc, gsems, isems, ssems):
    c = lax.axis_index("c")
    s = lax.axis_index("s")
    wid = s * NC + c
    pltpu.sync_copy(zeros_hbm.at[pl.ds(s * RPT, RPT)],
                    acc.at[pl.ds(s * RPT, RPT)])
    base = wid * EPW
    pltpu.sync_copy(src_hbm.at[pl.ds(base, EPW)], sidx)
    plsc.subcore_barrier()
    bufs = (rows0, rows1, rows2, rows3, rows4)
    dbufs = (di0, di1, di2, di3, di4)

    def stage(j, b):
        pltpu.async_copy(dst2_hbm.at[wid, j], dbufs[b], isems.at[b])
        pltpu.async_copy(
            table_hbm.at[sidx.at[pl.ds(j * ECHUNK, ECHUNK)]],
            bufs[b], gsems.at[b])

    def swait(j, b):
        pltpu.make_async_copy(dst2_hbm.at[wid, j], dbufs[b],
                              isems.at[b]).wait()
        pltpu.make_async_copy(
            table_hbm.at[sidx.at[pl.ds(j * ECHUNK, ECHUNK)]],
            bufs[b], gsems.at[b]).wait()

    def scat_wait(b):
        pltpu.make_async_copy(bufs[b], acc.at[dbufs[b]], ssems.at[b]).wait()

    for b in range(NBUF - 1):
        stage(b, b)

    def step(k, carry):
        for i in range(NBUF):
            j = k * NBUF + i
            swait(j, i)
            pltpu.async_copy(bufs[i], acc.at[dbufs[i]], ssems.at[i],
                             add=True)
            bp = (i + NBUF - 1) % NBUF

            @pl.when(j > 0)
            def _():
                scat_wait(bp)

            @pl.when(j + NBUF - 1 < ENCHUNK)
            def _():
                stage(j + NBUF - 1, bp)
        return carry

    lax.fori_loop(0, ENCHUNK // NBUF, step, 0)
    scat_wait((ENCHUNK - 1) % NBUF)
    plsc.subcore_barrier()
    pltpu.sync_copy(acc.at[pl.ds(s * RPT, RPT)],
                    out_hbm.at[c, pl.ds(s * RPT, RPT)])


def _deg_pass(dst, zeros80):
    return pl.kernel(
        _deg_body,
        out_type=jax.ShapeDtypeStruct((NC, HROW, 128), jnp.float32),
        mesh=_sc_mesh(),
        compiler_params=pltpu.CompilerParams(needs_layout_passes=False),
        scratch_types=[
            pltpu.VMEM((EPW,), jnp.int32),
            pltpu.VMEM((HROW, 128), jnp.float32),
            pltpu.VMEM((HROW,), jnp.int32),
            pltpu.VMEM_SHARED((HROW, 128), jnp.float32),
            pltpu.SemaphoreType.DMA,
        ],
    )(dst, zeros80)


def _edge_pass(d, table, src, dst2, zeros):
    return pl.kernel(
        functools.partial(_edge_body, d),
        out_type=jax.ShapeDtypeStruct((NC, N_PAD, d), jnp.float32),
        mesh=_sc_mesh(),
        scratch_types=(
            [pltpu.VMEM((EPW,), jnp.int32)]
            + [pltpu.VMEM((ECHUNK,), jnp.int32) for _ in range(NBUF)]
            + [pltpu.VMEM((ECHUNK, d), jnp.float32) for _ in range(NBUF)]
            + [pltpu.VMEM_SHARED((N_PAD, d), jnp.float32),
               pltpu.SemaphoreType.DMA((NBUF,)),
               pltpu.SemaphoreType.DMA((NBUF,)),
               pltpu.SemaphoreType.DMA((NBUF,))]
        ),
    )(table, src, dst2, zeros)


def _tca_body(x_ref, w_ref, degp_ref, g1_ref, dinv_ref):
    degf = (degp_ref[0] + degp_ref[1]).reshape(N_PAD)
    deg = degf[:N] + 1.0
    dinv = lax.rsqrt(deg)[:, None]
    h = jnp.dot(x_ref[...], w_ref[...], preferred_element_type=jnp.float32)
    g1_ref[...] = h * dinv
    dinv_ref[...] = dinv


def _bn(a, gamma, beta):
    mu = jnp.mean(a, axis=0, keepdims=True)
    var = jnp.mean((a - mu) * (a - mu), axis=0, keepdims=True)
    return gamma * (a - mu) * lax.rsqrt(var + 1e-5) + beta


def _tcb_body(s1_ref, g1t_ref, dinv_ref, b1_ref, g1_ref, be1_ref, w2_ref,
              g2t_ref):
    dinv = dinv_ref[...]
    pre = (s1_ref[0, :N] + s1_ref[1, :N] + g1t_ref[...]) * dinv + b1_ref[...]
    a = jnp.maximum(pre, 0.0)
    y = _bn(a, g1_ref[...], be1_ref[...])
    h2 = jnp.dot(y, w2_ref[...], preferred_element_type=jnp.float32)
    g2t_ref[:, :64] = h2 * dinv
    g2t_ref[:, 64:] = jnp.zeros((N, 64), jnp.float32)


def _tcc_body(s2_ref, g2t_ref, dinv_ref, b2_ref, g2_ref, be2_ref, batch_ref,
              fcw1_ref, fcb1_ref, g3_ref, be3_ref, fcw2_ref, fcb2_ref,
              fcw3_ref, fcb3_ref, out_ref):
    dinv = dinv_ref[...]
    pre = ((s2_ref[0, :N, :64] + s2_ref[1, :N, :64] + g2t_ref[:, :64]) * dinv
           + b2_ref[...])
    a = jnp.maximum(pre, 0.0)
    h = _bn(a, g2_ref[...], be2_ref[...])
    seg = lax.broadcasted_iota(jnp.int32, (G, N), 0)
    mask = (batch_ref[...][None, :] == seg).astype(jnp.float32)
    pooled = jnp.dot(mask, h, preferred_element_type=jnp.float32)
    r = jnp.maximum(
        jnp.dot(pooled, fcw1_ref[...], preferred_element_type=jnp.float32)
        + fcb1_ref[...], 0.0)
    r = _bn(r, g3_ref[...], be3_ref[...])
    r = jnp.maximum(
        jnp.dot(r, fcw2_ref[...], preferred_element_type=jnp.float32)
        + fcb2_ref[...], 0.0)
    logits = (jnp.dot(r, fcw3_ref[...], preferred_element_type=jnp.float32)
              + fcb3_ref[...])
    m = jnp.max(logits, axis=1, keepdims=True)
    z = logits - m
    out_ref[...] = z - jnp.log(jnp.sum(jnp.exp(z), axis=1, keepdims=True))


def _tc_call(body, out_shapes, *args):
    return pl.pallas_call(
        body,
        out_shape=out_shapes,
    )(*args)


def kernel(x, edge_index, batch, W1, b1, g1, be1, W2, b2, g2, be2,
           fcW1, fcb1, g3, be3, fcW2, fcb2, fcW3, fcb3):
    src = edge_index[0]
    dst = edge_index[1]
    dst2 = dst.reshape(NW, ENCHUNK, ECHUNK)
    zeros128 = jnp.zeros((N_PAD, D_IN), jnp.float32)

    degp = _deg_pass(dst, jnp.zeros((HROW, 128), jnp.float32))

    g1t, dinv = _tc_call(
        _tca_body,
        (jax.ShapeDtypeStruct((N, D_IN), jnp.float32),
         jax.ShapeDtypeStruct((N, 1), jnp.float32)),
        x, W1, degp)

    s1 = _edge_pass(D_IN, g1t, src, dst2, zeros128)

    g2t = _tc_call(
        _tcb_body,
        jax.ShapeDtypeStruct((N, D_IN), jnp.float32),
        s1, g1t, dinv, b1, g1, be1, W2)

    s2 = _edge_pass(D_IN, g2t, src, dst2, zeros128)

    out = _tc_call(
        _tcc_body,
        jax.ShapeDtypeStruct((G, 3), jnp.float32),
        s2, g2t, dinv, b2, g2, be2, batch,
        fcW1, fcb1, g3, be3, fcW2, fcb2, fcW3, fcb3)
    return out


# speedup vs baseline: 1.0372x; 1.0372x over previous
import functools

import jax
import jax.numpy as jnp
from jax import lax
from jax.experimental import pallas as pl
from jax.experimental.pallas import tpu as pltpu
from jax.experimental.pallas import tpu_sc as plsc

N = 10000
E = 320000
G = 64
D_IN = 128

NC = 2
NS = 16
NW = NC * NS
EPW = E // NW
CHUNK = 80
NCHUNK = EPW // CHUNK
ECHUNK = 40
ENCHUNK = EPW // ECHUNK
N_PAD = 10240
RPT = N_PAD // NS

def _sc_mesh():
    return plsc.VectorSubcoreMesh(
        core_axis_name="c", subcore_axis_name="s",
        num_cores=NC, num_subcores=NS)


HROW = N_PAD // 128


def _deg_body(dst_hbm, zeros_hbm, out_hbm, didx, hist, rid, acc, sem):
    c = lax.axis_index("c")
    s = lax.axis_index("s")
    wid = s * NC + c

    @pl.when(s == 0)
    def _():
        pltpu.sync_copy(zeros_hbm, acc)

    def zrow(i, carry):
        def zc(g, carry2):
            hist[i, pl.ds(g * 16, 16)] = jnp.zeros((16,), jnp.float32)
            return carry2
        return lax.fori_loop(0, 128 // 16, zc, carry)

    lax.fori_loop(0, HROW, zrow, 0)
    pltpu.sync_copy(dst_hbm.at[pl.ds(wid * EPW, EPW)], didx)
    for g in range(HROW // 16):
        rid[pl.ds(g * 16, 16)] = (
            lax.iota(jnp.int32, 16) + jnp.int32(g * 16))
    ones16 = jnp.ones((16,), jnp.float32)

    def grp(g, carry):
        iv = didx[pl.ds(g * 16, 16)]
        row = lax.shift_right_logical(iv, 7)
        col = lax.bitwise_and(iv, jnp.int32(127))
        plsc.addupdate_scatter(hist, [row, col], ones16)
        return carry

    lax.fori_loop(0, EPW // 16, grp, 0)
    plsc.subcore_barrier()
    pltpu.sync_copy(hist, acc.at[rid], add=True)
    plsc.subcore_barrier()

    @pl.when(s == 0)
    def _():
        pltpu.sync_copy(acc, out_hbm.at[c])


NBUF = 5


def _edge_body(d, table_hbm, src_hbm, dst2_hbm, zeros_hbm, out_hbm,
               sidx, di0, di1, di2, di3, di4,
               rows0, rows1, rows2, rows3, rows4,
               acc, gsems, isems):
    c = lax.axis_index("c")
    s = lax.axis_index("s")
    wid = s * NC + c
    pltpu.sync_copy(zeros_hbm.at[pl.ds(s * RPT, RPT)],
                    acc.at[pl.ds(s * RPT, RPT)])
    base = wid * EPW
    pltpu.sync_copy(src_hbm.at[pl.ds(base, EPW)], sidx)
    plsc.subcore_barrier()
    bufs = (rows0, rows1, rows2, rows3, rows4)
    dbufs = (di0, di1, di2, di3, di4)

    def stage(j, b):
        pltpu.async_copy(dst2_hbm.at[wid, j], dbufs[b], isems.at[b])
        pltpu.async_copy(
            table_hbm.at[sidx.at[pl.ds(j * ECHUNK, ECHUNK)]],
            bufs[b], gsems.at[b])

    def swait(j, b):
        pltpu.make_async_copy(dst2_hbm.at[wid, j], dbufs[b],
                              isems.at[b]).wait()
        pltpu.make_async_copy(
            table_hbm.at[sidx.at[pl.ds(j * ECHUNK, ECHUNK)]],
            bufs[b], gsems.at[b]).wait()

    for b in range(NBUF):
        stage(b, b)

    def step(k, carry):
        for i in range(NBUF):
            j = k * NBUF + i
            swait(j, i)
            pltpu.sync_copy(bufs[i], acc.at[dbufs[i]], add=True)
            stage(j + NBUF, i)
        return carry

    lax.fori_loop(0, ENCHUNK // NBUF - 1, step, 0)
    for i in range(NBUF):
        j = ENCHUNK - NBUF + i
        swait(j, i)
        pltpu.sync_copy(bufs[i], acc.at[dbufs[i]], add=True)
    plsc.subcore_barrier()
    pltpu.sync_copy(acc.at[pl.ds(s * RPT, RPT)],
                    out_hbm.at[c, pl.ds(s * RPT, RPT)])


def _deg_pass(dst, zeros80):
    return pl.kernel(
        _deg_body,
        out_type=jax.ShapeDtypeStruct((NC, HROW, 128), jnp.float32),
        mesh=_sc_mesh(),
        compiler_params=pltpu.CompilerParams(needs_layout_passes=False),
        scratch_types=[
            pltpu.VMEM((EPW,), jnp.int32),
            pltpu.VMEM((HROW, 128), jnp.float32),
            pltpu.VMEM((HROW,), jnp.int32),
            pltpu.VMEM_SHARED((HROW, 128), jnp.float32),
            pltpu.SemaphoreType.DMA,
        ],
    )(dst, zeros80)


def _edge_pass(d, table, src, dst2, zeros):
    return pl.kernel(
        functools.partial(_edge_body, d),
        out_type=jax.ShapeDtypeStruct((NC, N_PAD, d), jnp.float32),
        mesh=_sc_mesh(),
        scratch_types=(
            [pltpu.VMEM((EPW,), jnp.int32)]
            + [pltpu.VMEM((ECHUNK,), jnp.int32) for _ in range(NBUF)]
            + [pltpu.VMEM((ECHUNK, d), jnp.float32) for _ in range(NBUF)]
            + [pltpu.VMEM_SHARED((N_PAD, d), jnp.float32),
               pltpu.SemaphoreType.DMA((NBUF,)),
               pltpu.SemaphoreType.DMA((NBUF,))]
        ),
    )(table, src, dst2, zeros)


def _tca_body(x_ref, w_ref, degp_ref, g1_ref, dinv_ref):
    degf = (degp_ref[0] + degp_ref[1]).reshape(N_PAD)
    deg = degf[:N] + 1.0
    dinv = lax.rsqrt(deg)[:, None]
    h = jnp.dot(x_ref[...], w_ref[...], preferred_element_type=jnp.float32)
    g1_ref[...] = h * dinv
    dinv_ref[...] = dinv


def _bn(a, gamma, beta):
    mu = jnp.mean(a, axis=0, keepdims=True)
    var = jnp.mean((a - mu) * (a - mu), axis=0, keepdims=True)
    return gamma * (a - mu) * lax.rsqrt(var + 1e-5) + beta


def _tcb_body(s1_ref, g1t_ref, dinv_ref, b1_ref, g1_ref, be1_ref, w2_ref,
              g2t_ref):
    dinv = dinv_ref[...]
    pre = (s1_ref[0, :N] + s1_ref[1, :N] + g1t_ref[...]) * dinv + b1_ref[...]
    a = jnp.maximum(pre, 0.0)
    y = _bn(a, g1_ref[...], be1_ref[...])
    h2 = jnp.dot(y, w2_ref[...], preferred_element_type=jnp.float32)
    g2t_ref[:, :64] = h2 * dinv
    g2t_ref[:, 64:] = jnp.zeros((N, 64), jnp.float32)


def _tcc_body(s2_ref, g2t_ref, dinv_ref, b2_ref, g2_ref, be2_ref, batch_ref,
              fcw1_ref, fcb1_ref, g3_ref, be3_ref, fcw2_ref, fcb2_ref,
              fcw3_ref, fcb3_ref, out_ref):
    dinv = dinv_ref[...]
    pre = ((s2_ref[0, :N, :64] + s2_ref[1, :N, :64] + g2t_ref[:, :64]) * dinv
           + b2_ref[...])
    a = jnp.maximum(pre, 0.0)
    h = _bn(a, g2_ref[...], be2_ref[...])
    seg = lax.broadcasted_iota(jnp.int32, (G, N), 0)
    mask = (batch_ref[...][None, :] == seg).astype(jnp.float32)
    pooled = jnp.dot(mask, h, preferred_element_type=jnp.float32)
    r = jnp.maximum(
        jnp.dot(pooled, fcw1_ref[...], preferred_element_type=jnp.float32)
        + fcb1_ref[...], 0.0)
    r = _bn(r, g3_ref[...], be3_ref[...])
    r = jnp.maximum(
        jnp.dot(r, fcw2_ref[...], preferred_element_type=jnp.float32)
        + fcb2_ref[...], 0.0)
    logits = (jnp.dot(r, fcw3_ref[...], preferred_element_type=jnp.float32)
              + fcb3_ref[...])
    m = jnp.max(logits, axis=1, keepdims=True)
    z = logits - m
    out_ref[...] = z - jnp.log(jnp.sum(jnp.exp(z), axis=1, keepdims=True))


def _tc_call(body, out_shapes, *args):
    return pl.pallas_call(
        body,
        out_shape=out_shapes,
    )(*args)


def kernel(x, edge_index, batch, W1, b1, g1, be1, W2, b2, g2, be2,
           fcW1, fcb1, g3, be3, fcW2, fcb2, fcW3, fcb3):
    src = edge_index[0]
    dst = edge_index[1]
    dst2 = dst.reshape(NW, ENCHUNK, ECHUNK)
    zeros128 = jnp.zeros((N_PAD, D_IN), jnp.float32)

    degp = _deg_pass(dst, jnp.zeros((HROW, 128), jnp.float32))

    g1t, dinv = _tc_call(
        _tca_body,
        (jax.ShapeDtypeStruct((N, D_IN), jnp.float32),
         jax.ShapeDtypeStruct((N, 1), jnp.float32)),
        x, W1, degp)

    s1 = _edge_pass(D_IN, g1t, src, dst2, zeros128)

    g2t = _tc_call(
        _tcb_body,
        jax.ShapeDtypeStruct((N, D_IN), jnp.float32),
        s1, g1t, dinv, b1, g1, be1, W2)

    s2 = _edge_pass(D_IN, g2t, src, dst2, zeros128)

    out = _tc_call(
        _tcc_body,
        jax.ShapeDtypeStruct((G, 3), jnp.float32),
        s2, g2t, dinv, b2, g2, be2, batch,
        fcW1, fcb1, g3, be3, fcW2, fcb2, fcW3, fcb3)
    return out
